# Initial kernel scaffold; baseline (speedup 1.0000x reference)
import jax
import jax.numpy as jnp
from jax.experimental import pallas as pl


def kernel(x, edge_index, params):
    raise NotImplementedError("write your pallas kernel here")


# speedup vs baseline: 1.0000x; 1.0000x over previous
import jax
import jax.numpy as jnp
from jax.experimental import pallas as pl
from jax.experimental.pallas import tpu as pltpu

N_NODES = 10000
IN_FEAT = 128
HID = 128
HEADS = 8
DH = HID // HEADS
NL = 4


def _lrelu(v):
    return jnp.where(v > 0, v, 0.2 * v)


def _bn(h, g, b):
    return h * (g / jnp.sqrt(1.0 + 1e-5)) + b


def _cube_kernel(x_ref, o_ref):
    v = x_ref[...]
    o_ref[...] = v * v * v


def _cube(x):
    return pl.pallas_call(
        _cube_kernel,
        out_shape=jax.ShapeDtypeStruct(x.shape, x.dtype),
    )(x)


def kernel(x, edge_index, params):
    p = params
    N = x.shape[0]
    E = edge_index.shape[1]
    num_nodes = min(float(N), 1000000.0)
    num_edges = min(float(E), 1000000.0)
    avg_deg = min(float(E) / max(float(N), 1.0), 1000.0)
    gm = jnp.array([[num_nodes, num_edges, avg_deg]], dtype=jnp.float32)
    cent = x[:, :8]
    emb = x[:, 8:]
    ce = _bn(_lrelu(cent @ p['cw1'] + p['cb1']), p['cg1'], p['cbe1'])
    ee = _bn(_lrelu(emb @ p['ew1'] + p['eb1']), p['eg1'], p['ebe1'])
    h = jnp.concatenate([ce, ee], axis=1)
    h = _bn(_lrelu(h @ p['fw1'] + p['fb1']), p['fg1'], p['fbe1'])
    h = _bn(_lrelu(h @ p['fw2'] + p['fb2']), p['fg2'], p['fbe2'])
    attw = jax.nn.sigmoid(_lrelu(h @ p['faw1'] + p['fab1']) @ p['faw2'] + p['fab2'])
    h = h * attw
    h = h @ p['daw'] + p['dab']
    loop = jnp.arange(N, dtype=edge_index.dtype)
    src = jnp.concatenate([edge_index[0], loop])
    dst = jnp.concatenate([edge_index[1], loop])
    deg = jax.ops.segment_sum(jnp.ones(src.shape[0], jnp.float32), dst, num_segments=N)
    dinv = jnp.where(deg > 0, 1.0 / jnp.sqrt(deg), 0.0)
    norm = dinv[src] * dinv[dst]
    xs = []
    prev = h
    for i in range(NL):
        lp = p['layers'][i]
        xw = prev @ lp['gcnw']
        x_gcn = jax.ops.segment_sum(norm[:, None] * xw[src], dst, num_segments=N) + lp['gcnb']
        gw = (prev @ lp['gatw']).reshape(N, HEADS, DH)
        a_src = (gw * lp['gatas'][None]).sum(-1)
        a_dst = (gw * lp['gatad'][None]).sum(-1)
        e = _lrelu(a_src[src] + a_dst[dst])
        emax = jax.ops.segment_max(e, dst, num_segments=N)
        eexp = jnp.exp(e - emax[dst])
        den = jax.ops.segment_sum(eexp, dst, num_segments=N)
        alpha = eexp / (den[dst] + 1e-16)
        x_gat = jax.ops.segment_sum(alpha[:, :, None] * gw[src], dst, num_segments=N).reshape(N, HID) + lp['gatb']
        cur = _lrelu(_bn(x_gcn + x_gat, lp['bng'], lp['bnb']))
        base_gate = jax.nn.sigmoid(_lrelu(cur @ lp['gw1'] + lp['gb1']) @ lp['gw2'] + lp['gb2'])
        gf = jax.nn.sigmoid(_lrelu(gm @ lp['gmw1'] + lp['gmb1']) @ lp['gmw2'] + lp['gmb2'])
        depth = jax.nn.sigmoid(jnp.float32(i) / float(max(NL, 1)) * jnp.ones_like(base_gate))
        fg = base_gate * gf * depth
        cur = fg * cur + (1.0 - fg) * (prev + lp['denc'])
        prev = cur
        xs.append(cur)
    hcat = jnp.concatenate(xs, axis=1)
    h2 = _lrelu(hcat @ p['lin1w'] + p['lin1b'])
    m = _bn(_lrelu(h2 @ p['mw1'] + p['mb1']), p['mg1'], p['mbe1'])
    m = _bn(_lrelu(m @ p['mw2'] + p['mb2']), p['mg2'], p['mbe2'])
    raw = jax.nn.sigmoid(m @ p['mw3'] + p['mb3']).squeeze(-1)
    scaled = raw / p['temp'][0]
    sorted_desc = jnp.sort(scaled)[::-1]
    enhanced = jax.nn.softmax(scaled, axis=0)
    k = max(1, int(N * 0.2))
    thr = sorted_desc[k - 1]
    mask = (scaled >= thr).astype(jnp.float32)
    boosted = enhanced * (mask + 0.1 * (1.0 - mask))
    pad = (-boosted.shape[0]) % 128
    bp = jnp.pad(boosted, (0, pad)).reshape(-1, 128)
    out = _cube(bp).reshape(-1)[:boosted.shape[0]]
    return out


# speedup vs baseline: 16.7665x; 16.7660x over previous
import functools

import jax
import jax.numpy as jnp
from jax import lax
from jax.experimental import pallas as pl
from jax.experimental.pallas import tpu as pltpu
from jax.experimental.pallas import tpu_sc as plsc

N = 10000
NP = 10016
RPT = NP // 16
HID = 128
HEADS = 8
DH = 16
NL = 4
HB = 9
B = 128
NBATCH = 84
EPT = B * NBATCH
EP = 16 * EPT
NB2 = NBATCH // 2

def _mesh():
    return plsc.VectorSubcoreMesh(
        core_axis_name="c", subcore_axis_name="s", num_cores=2, num_subcores=16)


_SC_PARAMS = pltpu.CompilerParams(use_tc_tiling_on_sc=False)


def _deg_body(ds_hbm, ones_hbm, zero_hbm, out_hbm, idx_v, ones_v, dacc):
    cid = lax.axis_index("c")
    sid = lax.axis_index("s")
    pltpu.sync_copy(zero_hbm.at[pl.ds(sid * RPT, RPT)], dacc.at[pl.ds(sid * RPT, RPT)])
    pltpu.sync_copy(ones_hbm, ones_v)
    plsc.subcore_barrier()
    base0 = (cid * 16 + sid) * (EP // 32)

    def batch(bi, carry):
        pltpu.sync_copy(ds_hbm.at[pl.ds(base0 + bi * B, B)], idx_v)
        pltpu.sync_copy(ones_v, dacc.at[idx_v], add=True)
        return carry

    lax.fori_loop(0, EP // 32 // B, batch, 0)
    plsc.subcore_barrier()
    pltpu.sync_copy(dacc.at[pl.ds(sid * RPT, RPT)],
                    out_hbm.at[cid, pl.ds(sid * RPT, RPT)])


def _deg_call(*args):
    return pl.kernel(
        _deg_body,
        out_type=jax.ShapeDtypeStruct((2, NP, 16), jnp.float32),
        mesh=_mesh(),
        compiler_params=_SC_PARAMS,
        scratch_types=[
            pltpu.VMEM((B,), jnp.int32),
            pltpu.VMEM((B, 16), jnp.float32),
            pltpu.VMEM_SHARED((NP, 16), jnp.float32),
        ],
    )(*args)


def _edge_body(t_hbm, ws_hbm, wd_hbm, gs_hbm, gwd_hbm, ds_hbm, zero_hbm,
               out_hbm, gs_v, gwd_v, ds_v, rows_v, ws_v, wd_v, pay_v, sem, acc):
    cid = lax.axis_index("c")
    sid = lax.axis_index("s")
    is_gcn = cid == 0
    pltpu.sync_copy(zero_hbm.at[pl.ds(sid * RPT, RPT)], acc.at[pl.ds(sid * RPT, RPT)])
    plsc.subcore_barrier()
    base0 = sid * EPT

    def batch(bi, carry):
        base = base0 + bi * B
        pltpu.sync_copy(gs_hbm.at[cid, pl.ds(base, B)], gs_v)
        pltpu.sync_copy(gwd_hbm.at[cid, pl.ds(base, B)], gwd_v)
        pltpu.sync_copy(ds_hbm.at[pl.ds(base, B)], ds_v)
        c1 = pltpu.async_copy(t_hbm.at[gs_v], rows_v, sem)
        c2 = pltpu.async_copy(ws_hbm.at[gs_v], ws_v, sem)
        c3 = pltpu.async_copy(wd_hbm.at[gwd_v], wd_v, sem)
        c1.wait()
        c2.wait()
        c3.wait()

        def edge(e, ecarry):
            a = ws_v[e]
            b = wd_v[e]
            s = a + b
            gat = jnp.exp(jnp.where(s > 0.0, s, 0.2 * s))
            w = jnp.where(is_gcn, a * b, gat)
            pay_v[e, 8] = w
            for h in range(HEADS):
                wh = lax.squeeze(lax.slice(w, (h,), (h + 1,)), (0,))
                pay_v[e, h] = rows_v[e, h] * wh
            return ecarry

        lax.fori_loop(0, B, edge, 0)
        pltpu.sync_copy(pay_v, acc.at[ds_v], add=True)
        return carry

    lax.fori_loop(0, NBATCH, batch, 0)
    plsc.subcore_barrier()
    pltpu.sync_copy(acc.at[pl.ds(sid * RPT, RPT)],
                    out_hbm.at[cid, pl.ds(sid * RPT, RPT)])


def _edge_call(*args):
    return pl.kernel(
        _edge_body,
        out_type=jax.ShapeDtypeStruct((2, NP, HB, 16), jnp.float32),
        mesh=_mesh(),
        compiler_params=_SC_PARAMS,
        scratch_types=[
            pltpu.VMEM((B,), jnp.int32),
            pltpu.VMEM((B,), jnp.int32),
            pltpu.VMEM((B,), jnp.int32),
            pltpu.VMEM((B, HEADS, 16), jnp.float32),
            pltpu.VMEM((B, 16), jnp.float32),
            pltpu.VMEM((B, 16), jnp.float32),
            pltpu.VMEM((B, HB, 16), jnp.float32),
            pltpu.SemaphoreType.DMA,
            pltpu.VMEM_SHARED((NP, HB, 16), jnp.float32),
        ],
    )(*args)


def _lrelu(v):
    return jnp.where(v > 0, v, 0.2 * v)


def _bn(h, g, b):
    return h * (g / jnp.sqrt(1.0 + 1e-5)) + b


def kernel(x, edge_index, params):
    p = params
    E = edge_index.shape[1]
    num_nodes = min(float(N), 1000000.0)
    num_edges = min(float(E), 1000000.0)
    avg_deg = min(float(E) / max(float(N), 1.0), 1000.0)
    gm = jnp.array([[num_nodes, num_edges, avg_deg]], dtype=jnp.float32)

    loop = jnp.arange(N, dtype=jnp.int32)
    src = jnp.concatenate([edge_index[0].astype(jnp.int32), loop])
    dst = jnp.concatenate([edge_index[1].astype(jnp.int32), loop])
    npad = EP - src.shape[0]
    src_p = jnp.pad(src, (0, npad))
    dst_p = jnp.pad(dst, (0, npad), constant_values=N)
    gs = jnp.stack([src_p, src_p + N])
    gwd = jnp.stack([dst_p, dst_p + N])
    zero_acc = jnp.zeros((NP, HB, 16), jnp.float32)
    zero_deg = jnp.zeros((NP, 16), jnp.float32)
    ones16 = jnp.ones((B, 16), jnp.float32)

    dout = _deg_call(dst_p, ones16, zero_deg)
    deg = dout[0, :N, 0] + dout[1, :N, 0]
    dinv = jnp.where(deg > 0, 1.0 / jnp.sqrt(deg), 0.0)
    dinv16 = jnp.broadcast_to(dinv[:, None], (N, 16))

    cent = x[:, :8]
    emb = x[:, 8:]
    ce = _bn(_lrelu(cent @ p['cw1'] + p['cb1']), p['cg1'], p['cbe1'])
    ee = _bn(_lrelu(emb @ p['ew1'] + p['eb1']), p['eg1'], p['ebe1'])
    h = jnp.concatenate([ce, ee], axis=1)
    h = _bn(_lrelu(h @ p['fw1'] + p['fb1']), p['fg1'], p['fbe1'])
    h = _bn(_lrelu(h @ p['fw2'] + p['fb2']), p['fg2'], p['fbe2'])
    attw = jax.nn.sigmoid(_lrelu(h @ p['faw1'] + p['fab1']) @ p['faw2'] + p['fab2'])
    h = h * attw
    h = h @ p['daw'] + p['dab']

    pad16 = jnp.zeros((N, 8), jnp.float32)
    wdpad = jnp.zeros((16, 16), jnp.float32)
    xs = []
    prev = h
    for i in range(NL):
        lp = p['layers'][i]
        xw = prev @ lp['gcnw']
        gwm = prev @ lp['gatw']
        a_src = (gwm.reshape(N, HEADS, DH) * lp['gatas'][None]).sum(-1)
        a_dst = (gwm.reshape(N, HEADS, DH) * lp['gatad'][None]).sum(-1)
        tbl = jnp.concatenate([xw, gwm]).reshape(2 * N, HEADS, 16)
        ws_t = jnp.concatenate(
            [dinv16, jnp.concatenate([a_src, pad16], axis=1)])
        wd_t = jnp.concatenate(
            [dinv16, jnp.concatenate([a_dst, pad16], axis=1), wdpad])
        acc = _edge_call(tbl, ws_t, wd_t, gs, gwd, dst_p, zero_acc)
        a0 = acc[0, :N].reshape(N, HB * 16)
        a1 = acc[1, :N]
        x_gcn = a0[:, :HID] + lp['gcnb']
        den = a1[:, 8, :HEADS]
        x_gat = (a1[:, :HEADS] / (den[..., None] + 1e-16)).reshape(N, HID) + lp['gatb']
        cur = _lrelu(_bn(x_gcn + x_gat, lp['bng'], lp['bnb']))
        base_gate = jax.nn.sigmoid(_lrelu(cur @ lp['gw1'] + lp['gb1']) @ lp['gw2'] + lp['gb2'])
        gf = jax.nn.sigmoid(_lrelu(gm @ lp['gmw1'] + lp['gmb1']) @ lp['gmw2'] + lp['gmb2'])
        depth = jax.nn.sigmoid(jnp.float32(i) / float(max(NL, 1)) * jnp.ones_like(base_gate))
        fg = base_gate * gf * depth
        cur = fg * cur + (1.0 - fg) * (prev + lp['denc'])
        prev = cur
        xs.append(cur)

    hcat = jnp.concatenate(xs, axis=1)
    h2 = _lrelu(hcat @ p['lin1w'] + p['lin1b'])
    m = _bn(_lrelu(h2 @ p['mw1'] + p['mb1']), p['mg1'], p['mbe1'])
    m = _bn(_lrelu(m @ p['mw2'] + p['mb2']), p['mg2'], p['mbe2'])
    raw = jax.nn.sigmoid(m @ p['mw3'] + p['mb3']).squeeze(-1)
    scaled = raw / p['temp'][0]
    sorted_desc = jnp.sort(scaled)[::-1]
    enhanced = jax.nn.softmax(scaled, axis=0)
    k = max(1, int(N * 0.2))
    thr = sorted_desc[k - 1]
    mask = (scaled >= thr).astype(jnp.float32)
    boosted = enhanced * (mask + 0.1 * (1.0 - mask))
    return boosted ** 3.0


# speedup vs baseline: 17.0159x; 1.0149x over previous
import functools

import jax
import jax.numpy as jnp
from jax import lax
from jax.experimental import pallas as pl
from jax.experimental.pallas import tpu as pltpu
from jax.experimental.pallas import tpu_sc as plsc

N = 10000
NP = 10016
RPT = NP // 16
HID = 128
HEADS = 8
DH = 16
NL = 4
HB = 9
B = 128
NBATCH = 84
EPT = B * NBATCH
EP = 16 * EPT
NB2 = NBATCH // 2

def _mesh():
    return plsc.VectorSubcoreMesh(
        core_axis_name="c", subcore_axis_name="s", num_cores=2, num_subcores=16)


_SC_PARAMS = pltpu.CompilerParams(use_tc_tiling_on_sc=False)


def _deg_body(ds_hbm, ones_hbm, zero_hbm, out_hbm, idx_v, ones_v, dacc):
    cid = lax.axis_index("c")
    sid = lax.axis_index("s")
    pltpu.sync_copy(zero_hbm.at[pl.ds(sid * RPT, RPT)], dacc.at[pl.ds(sid * RPT, RPT)])
    pltpu.sync_copy(ones_hbm, ones_v)
    plsc.subcore_barrier()
    base0 = (cid * 16 + sid) * (EP // 32)

    def batch(bi, carry):
        pltpu.sync_copy(ds_hbm.at[pl.ds(base0 + bi * B, B)], idx_v)
        pltpu.sync_copy(ones_v, dacc.at[idx_v], add=True)
        return carry

    lax.fori_loop(0, EP // 32 // B, batch, 0)
    plsc.subcore_barrier()
    pltpu.sync_copy(dacc.at[pl.ds(sid * RPT, RPT)],
                    out_hbm.at[cid, pl.ds(sid * RPT, RPT)])


def _deg_call(*args):
    return pl.kernel(
        _deg_body,
        out_type=jax.ShapeDtypeStruct((2, NP, 16), jnp.float32),
        mesh=_mesh(),
        compiler_params=_SC_PARAMS,
        scratch_types=[
            pltpu.VMEM((B,), jnp.int32),
            pltpu.VMEM((B, 16), jnp.float32),
            pltpu.VMEM_SHARED((NP, 16), jnp.float32),
        ],
    )(*args)


def _edge_body(t_hbm, ws_hbm, wd_hbm, gs_hbm, gwd_hbm, ds_hbm, zero_hbm,
               out_hbm, gs_v, gwd_v, ds_v, rows_v, ws_v, wd_v, pay_v, sem, acc):
    cid = lax.axis_index("c")
    sid = lax.axis_index("s")
    is_gcn = cid == 0
    pltpu.sync_copy(zero_hbm.at[pl.ds(sid * RPT, RPT)], acc.at[pl.ds(sid * RPT, RPT)])
    plsc.subcore_barrier()
    base0 = sid * EPT

    def batch(bi, carry):
        base = base0 + bi * B
        pltpu.sync_copy(gs_hbm.at[cid, pl.ds(base, B)], gs_v)
        pltpu.sync_copy(gwd_hbm.at[cid, pl.ds(base, B)], gwd_v)
        pltpu.sync_copy(ds_hbm.at[pl.ds(base, B)], ds_v)
        c1 = pltpu.async_copy(t_hbm.at[gs_v], rows_v, sem)
        c2 = pltpu.async_copy(ws_hbm.at[gs_v], ws_v, sem)
        c3 = pltpu.async_copy(wd_hbm.at[gwd_v], wd_v, sem)
        c1.wait()
        c2.wait()
        c3.wait()

        def edge_gcn(e, ecarry):
            w = ws_v[e] * wd_v[e]
            pay_v[e, 8] = w
            wh = w[0]
            for h in range(HEADS):
                pay_v[e, h] = rows_v[e, h] * wh
            return ecarry

        def edge_gat(e, ecarry):
            s = ws_v[e] + wd_v[e]
            w = jnp.exp(jnp.where(s > 0.0, s, 0.2 * s))
            pay_v[e, 8] = w
            for h in range(HEADS):
                pay_v[e, h] = rows_v[e, h] * w[h]
            return ecarry

        @pl.when(is_gcn)
        def _():
            lax.fori_loop(0, B, edge_gcn, 0, unroll=4)

        @pl.when(jnp.logical_not(is_gcn))
        def _():
            lax.fori_loop(0, B, edge_gat, 0, unroll=4)
        pltpu.sync_copy(pay_v, acc.at[ds_v], add=True)
        return carry

    lax.fori_loop(0, NBATCH, batch, 0)
    plsc.subcore_barrier()
    pltpu.sync_copy(acc.at[pl.ds(sid * RPT, RPT)],
                    out_hbm.at[cid, pl.ds(sid * RPT, RPT)])


def _edge_call(*args):
    return pl.kernel(
        _edge_body,
        out_type=jax.ShapeDtypeStruct((2, NP, HB, 16), jnp.float32),
        mesh=_mesh(),
        compiler_params=_SC_PARAMS,
        scratch_types=[
            pltpu.VMEM((B,), jnp.int32),
            pltpu.VMEM((B,), jnp.int32),
            pltpu.VMEM((B,), jnp.int32),
            pltpu.VMEM((B, HEADS, 16), jnp.float32),
            pltpu.VMEM((B, 16), jnp.float32),
            pltpu.VMEM((B, 16), jnp.float32),
            pltpu.VMEM((B, HB, 16), jnp.float32),
            pltpu.SemaphoreType.DMA,
            pltpu.VMEM_SHARED((NP, HB, 16), jnp.float32),
        ],
    )(*args)


def _lrelu(v):
    return jnp.where(v > 0, v, 0.2 * v)


def _bn(h, g, b):
    return h * (g / jnp.sqrt(1.0 + 1e-5)) + b


def kernel(x, edge_index, params):
    p = params
    E = edge_index.shape[1]
    num_nodes = min(float(N), 1000000.0)
    num_edges = min(float(E), 1000000.0)
    avg_deg = min(float(E) / max(float(N), 1.0), 1000.0)
    gm = jnp.array([[num_nodes, num_edges, avg_deg]], dtype=jnp.float32)

    loop = jnp.arange(N, dtype=jnp.int32)
    src = jnp.concatenate([edge_index[0].astype(jnp.int32), loop])
    dst = jnp.concatenate([edge_index[1].astype(jnp.int32), loop])
    npad = EP - src.shape[0]
    src_p = jnp.pad(src, (0, npad))
    dst_p = jnp.pad(dst, (0, npad), constant_values=N)
    gs = jnp.stack([src_p, src_p + N])
    gwd = jnp.stack([dst_p, dst_p + N])
    zero_acc = jnp.zeros((NP, HB, 16), jnp.float32)
    zero_deg = jnp.zeros((NP, 16), jnp.float32)
    ones16 = jnp.ones((B, 16), jnp.float32)

    dout = _deg_call(dst_p, ones16, zero_deg)
    deg = dout[0, :N, 0] + dout[1, :N, 0]
    dinv = jnp.where(deg > 0, 1.0 / jnp.sqrt(deg), 0.0)
    dinv16 = jnp.broadcast_to(dinv[:, None], (N, 16))

    cent = x[:, :8]
    emb = x[:, 8:]
    ce = _bn(_lrelu(cent @ p['cw1'] + p['cb1']), p['cg1'], p['cbe1'])
    ee = _bn(_lrelu(emb @ p['ew1'] + p['eb1']), p['eg1'], p['ebe1'])
    h = jnp.concatenate([ce, ee], axis=1)
    h = _bn(_lrelu(h @ p['fw1'] + p['fb1']), p['fg1'], p['fbe1'])
    h = _bn(_lrelu(h @ p['fw2'] + p['fb2']), p['fg2'], p['fbe2'])
    attw = jax.nn.sigmoid(_lrelu(h @ p['faw1'] + p['fab1']) @ p['faw2'] + p['fab2'])
    h = h * attw
    h = h @ p['daw'] + p['dab']

    pad16 = jnp.zeros((N, 8), jnp.float32)
    wdpad = jnp.zeros((16, 16), jnp.float32)
    xs = []
    prev = h
    for i in range(NL):
        lp = p['layers'][i]
        xw = prev @ lp['gcnw']
        gwm = prev @ lp['gatw']
        a_src = (gwm.reshape(N, HEADS, DH) * lp['gatas'][None]).sum(-1)
        a_dst = (gwm.reshape(N, HEADS, DH) * lp['gatad'][None]).sum(-1)
        tbl = jnp.concatenate([xw, gwm]).reshape(2 * N, HEADS, 16)
        ws_t = jnp.concatenate(
            [dinv16, jnp.concatenate([a_src, pad16], axis=1)])
        wd_t = jnp.concatenate(
            [dinv16, jnp.concatenate([a_dst, pad16], axis=1), wdpad])
        acc = _edge_call(tbl, ws_t, wd_t, gs, gwd, dst_p, zero_acc)
        a0 = acc[0, :N].reshape(N, HB * 16)
        a1 = acc[1, :N]
        x_gcn = a0[:, :HID] + lp['gcnb']
        den = a1[:, 8, :HEADS]
        x_gat = (a1[:, :HEADS] / (den[..., None] + 1e-16)).reshape(N, HID) + lp['gatb']
        cur = _lrelu(_bn(x_gcn + x_gat, lp['bng'], lp['bnb']))
        base_gate = jax.nn.sigmoid(_lrelu(cur @ lp['gw1'] + lp['gb1']) @ lp['gw2'] + lp['gb2'])
        gf = jax.nn.sigmoid(_lrelu(gm @ lp['gmw1'] + lp['gmb1']) @ lp['gmw2'] + lp['gmb2'])
        depth = jax.nn.sigmoid(jnp.float32(i) / float(max(NL, 1)) * jnp.ones_like(base_gate))
        fg = base_gate * gf * depth
        cur = fg * cur + (1.0 - fg) * (prev + lp['denc'])
        prev = cur
        xs.append(cur)

    hcat = jnp.concatenate(xs, axis=1)
    h2 = _lrelu(hcat @ p['lin1w'] + p['lin1b'])
    m = _bn(_lrelu(h2 @ p['mw1'] + p['mb1']), p['mg1'], p['mbe1'])
    m = _bn(_lrelu(m @ p['mw2'] + p['mb2']), p['mg2'], p['mbe2'])
    raw = jax.nn.sigmoid(m @ p['mw3'] + p['mb3']).squeeze(-1)
    scaled = raw / p['temp'][0]
    sorted_desc = jnp.sort(scaled)[::-1]
    enhanced = jax.nn.softmax(scaled, axis=0)
    k = max(1, int(N * 0.2))
    thr = sorted_desc[k - 1]
    mask = (scaled >= thr).astype(jnp.float32)
    boosted = enhanced * (mask + 0.1 * (1.0 - mask))
    return boosted ** 3.0


# speedup vs baseline: 34.1814x; 2.0088x over previous
import functools

import jax
import jax.numpy as jnp
from jax import lax
from jax.experimental import pallas as pl
from jax.experimental.pallas import tpu as pltpu
from jax.experimental.pallas import tpu_sc as plsc

N = 10000
NP = 10016
RPT = NP // 16
HID = 128
HEADS = 8
DH = 16
NL = 4
HB = 9
B = 112
NBATCH = 96
EPT = B * NBATCH
EP = 16 * EPT
DB = 128

def _mesh():
    return plsc.VectorSubcoreMesh(
        core_axis_name="c", subcore_axis_name="s", num_cores=2, num_subcores=16)


_SC_PARAMS = pltpu.CompilerParams(use_tc_tiling_on_sc=False)


def _deg_body(ds_hbm, ones_hbm, zero_hbm, out_hbm, idx_v, ones_v, dacc):
    cid = lax.axis_index("c")
    sid = lax.axis_index("s")
    pltpu.sync_copy(zero_hbm.at[pl.ds(sid * RPT, RPT)], dacc.at[pl.ds(sid * RPT, RPT)])
    pltpu.sync_copy(ones_hbm, ones_v)
    plsc.subcore_barrier()
    base0 = (cid * 16 + sid) * (EP // 32)

    def batch(bi, carry):
        pltpu.sync_copy(ds_hbm.at[pl.ds(base0 + bi * DB, DB)], idx_v)
        pltpu.sync_copy(ones_v, dacc.at[idx_v], add=True)
        return carry

    lax.fori_loop(0, EP // 32 // DB, batch, 0)
    plsc.subcore_barrier()
    pltpu.sync_copy(dacc.at[pl.ds(sid * RPT, RPT)],
                    out_hbm.at[cid, pl.ds(sid * RPT, RPT)])


def _deg_call(*args):
    return pl.kernel(
        _deg_body,
        out_type=jax.ShapeDtypeStruct((2, NP, 16), jnp.float32),
        mesh=_mesh(),
        compiler_params=_SC_PARAMS,
        scratch_types=[
            pltpu.VMEM((DB,), jnp.int32),
            pltpu.VMEM((DB, 16), jnp.float32),
            pltpu.VMEM_SHARED((NP, 16), jnp.float32),
        ],
    )(*args)


def _edge_body(t_hbm, ws_hbm, wd_hbm, idx_hbm, zero8_hbm, zerod_hbm,
               out8_hbm, outd_hbm, ib0, ib1, r0, r1, s0, s1, d0, d1,
               isem0, isem1, gsem0, gsem1, acc8, accd):
    cid = lax.axis_index("c")
    sid = lax.axis_index("s")
    is_gcn = cid == 0
    ib = (ib0, ib1)
    rows = (r0, r1)
    wsv = (s0, s1)
    wdv = (d0, d1)
    isem = (isem0, isem1)
    gsem = (gsem0, gsem1)
    pltpu.sync_copy(zero8_hbm.at[pl.ds(sid * RPT, RPT)],
                    acc8.at[pl.ds(sid * RPT, RPT)])
    pltpu.sync_copy(zerod_hbm.at[pl.ds(sid * RPT, RPT)],
                    accd.at[pl.ds(sid * RPT, RPT)])

    def fire_i(i, b):
        pltpu.async_copy(idx_hbm.at[cid, sid, i], ib[b], isem[b])

    def wait_i(b):
        pltpu.make_async_copy(idx_hbm.at[cid, sid, 0], ib[b], isem[b]).wait()

    def fire_g(b):
        pltpu.async_copy(t_hbm.at[ib[b].at[0]], rows[b], gsem[b])
        pltpu.async_copy(ws_hbm.at[ib[b].at[0]], wsv[b], gsem[b])
        pltpu.async_copy(wd_hbm.at[ib[b].at[1]], wdv[b], gsem[b])

    def wait_g(b):
        pltpu.make_async_copy(t_hbm.at[ib[b].at[0]], rows[b], gsem[b]).wait()
        pltpu.make_async_copy(ws_hbm.at[ib[b].at[0]], wsv[b], gsem[b]).wait()
        pltpu.make_async_copy(wd_hbm.at[ib[b].at[1]], wdv[b], gsem[b]).wait()

    def compute(b):
        rv = rows[b]
        sv = wsv[b]
        dv = wdv[b]

        def edge_gcn(e, ecarry):
            w = sv[e] * dv[e]
            sv[e] = w
            wh = w[0]
            for h in range(HEADS):
                rv[e, h] = rv[e, h] * wh
            return ecarry

        def edge_gat(e, ecarry):
            s = sv[e] + dv[e]
            w = jnp.exp(jnp.where(s > 0.0, s, 0.2 * s))
            sv[e] = w
            for h in range(HEADS):
                rv[e, h] = rv[e, h] * w[h]
            return ecarry

        @pl.when(is_gcn)
        def _():
            lax.fori_loop(0, B, edge_gcn, 0, unroll=4)

        @pl.when(jnp.logical_not(is_gcn))
        def _():
            lax.fori_loop(0, B, edge_gat, 0, unroll=4)

    fire_i(0, 0)
    plsc.subcore_barrier()
    wait_i(0)
    fire_g(0)
    fire_i(1, 1)

    def batch_pair(ip, carry):
        for b in (0, 1):
            i = ip * 2 + b

            @pl.when(i + 1 <= NBATCH)
            def _():
                wait_i(1 - b)

            @pl.when(i + 1 < NBATCH)
            def _():
                fire_g(1 - b)

            wait_g(b)

            @pl.when(i + 2 <= NBATCH)
            def _():
                fire_i(i + 2, b)

            compute(b)
            pltpu.sync_copy(rows[b], acc8.at[ib[1 - b].at[2]], add=True)
            pltpu.sync_copy(wsv[b], accd.at[ib[1 - b].at[2]], add=True)
        return carry

    lax.fori_loop(0, NBATCH // 2, batch_pair, 0)
    plsc.subcore_barrier()
    pltpu.sync_copy(acc8.at[pl.ds(sid * RPT, RPT)],
                    out8_hbm.at[cid, pl.ds(sid * RPT, RPT)])
    pltpu.sync_copy(accd.at[pl.ds(sid * RPT, RPT)],
                    outd_hbm.at[cid, pl.ds(sid * RPT, RPT)])


def _edge_call(*args):
    return pl.kernel(
        _edge_body,
        out_type=(jax.ShapeDtypeStruct((2, NP, HEADS, 16), jnp.float32),
                  jax.ShapeDtypeStruct((2, NP, 16), jnp.float32)),
        mesh=_mesh(),
        compiler_params=_SC_PARAMS,
        scratch_types=[
            pltpu.VMEM((3, B), jnp.int32),
            pltpu.VMEM((3, B), jnp.int32),
            pltpu.VMEM((B, HEADS, 16), jnp.float32),
            pltpu.VMEM((B, HEADS, 16), jnp.float32),
            pltpu.VMEM((B, 16), jnp.float32),
            pltpu.VMEM((B, 16), jnp.float32),
            pltpu.VMEM((B, 16), jnp.float32),
            pltpu.VMEM((B, 16), jnp.float32),
            pltpu.SemaphoreType.DMA,
            pltpu.SemaphoreType.DMA,
            pltpu.SemaphoreType.DMA,
            pltpu.SemaphoreType.DMA,
            pltpu.VMEM_SHARED((NP, HEADS, 16), jnp.float32),
            pltpu.VMEM_SHARED((NP, 16), jnp.float32),
        ],
    )(*args)


def _lrelu(v):
    return jnp.where(v > 0, v, 0.2 * v)


def _bn(h, g, b):
    return h * (g / jnp.sqrt(1.0 + 1e-5)) + b


def kernel(x, edge_index, params):
    p = params
    E = edge_index.shape[1]
    num_nodes = min(float(N), 1000000.0)
    num_edges = min(float(E), 1000000.0)
    avg_deg = min(float(E) / max(float(N), 1.0), 1000.0)
    gm = jnp.array([[num_nodes, num_edges, avg_deg]], dtype=jnp.float32)

    loop = jnp.arange(N, dtype=jnp.int32)
    src = jnp.concatenate([edge_index[0].astype(jnp.int32), loop])
    dst = jnp.concatenate([edge_index[1].astype(jnp.int32), loop])
    npad = EP - src.shape[0]
    src_p = jnp.pad(src, (0, npad))
    dst_p = jnp.pad(dst, (0, npad), constant_values=N)
    gs = jnp.stack([src_p, src_p + N])
    gwd = jnp.stack([dst_p, dst_p + N])
    gg = jnp.stack([gs.reshape(2, 16, NBATCH, B),
                    gwd.reshape(2, 16, NBATCH, B)], axis=3)
    gg = jnp.pad(gg, ((0, 0), (0, 0), (0, 1), (0, 0), (0, 0)))
    ds_r = dst_p.reshape(16, NBATCH, B)
    ds_sh = jnp.concatenate(
        [jnp.full((16, 1, B), N, jnp.int32), ds_r], axis=1)
    ds_sh = jnp.broadcast_to(ds_sh[None, :, :, None], (2, 16, NBATCH + 1, 1, B))
    idxc = jnp.concatenate([gg, ds_sh], axis=3)
    zero8 = jnp.zeros((NP, HEADS, 16), jnp.float32)
    zerod = jnp.zeros((NP, 16), jnp.float32)
    zero_deg = jnp.zeros((NP, 16), jnp.float32)
    ones16 = jnp.ones((DB, 16), jnp.float32)

    dout = _deg_call(dst_p, ones16, zero_deg)
    deg = dout[0, :N, 0] + dout[1, :N, 0]
    dinv = jnp.where(deg > 0, 1.0 / jnp.sqrt(deg), 0.0)
    dinv16 = jnp.broadcast_to(dinv[:, None], (N, 16))

    cent = x[:, :8]
    emb = x[:, 8:]
    ce = _bn(_lrelu(cent @ p['cw1'] + p['cb1']), p['cg1'], p['cbe1'])
    ee = _bn(_lrelu(emb @ p['ew1'] + p['eb1']), p['eg1'], p['ebe1'])
    h = jnp.concatenate([ce, ee], axis=1)
    h = _bn(_lrelu(h @ p['fw1'] + p['fb1']), p['fg1'], p['fbe1'])
    h = _bn(_lrelu(h @ p['fw2'] + p['fb2']), p['fg2'], p['fbe2'])
    attw = jax.nn.sigmoid(_lrelu(h @ p['faw1'] + p['fab1']) @ p['faw2'] + p['fab2'])
    h = h * attw
    h = h @ p['daw'] + p['dab']

    pad16 = jnp.zeros((N, 8), jnp.float32)
    wdpad = jnp.zeros((16, 16), jnp.float32)
    xs = []
    prev = h
    for i in range(NL):
        lp = p['layers'][i]
        xw = prev @ lp['gcnw']
        gwm = prev @ lp['gatw']
        a_src = (gwm.reshape(N, HEADS, DH) * lp['gatas'][None]).sum(-1)
        a_dst = (gwm.reshape(N, HEADS, DH) * lp['gatad'][None]).sum(-1)
        tbl = jnp.concatenate([xw, gwm]).reshape(2 * N, HEADS, 16)
        ws_t = jnp.concatenate(
            [dinv16, jnp.concatenate([a_src, pad16], axis=1)])
        wd_t = jnp.concatenate(
            [dinv16, jnp.concatenate([a_dst, pad16], axis=1), wdpad])
        acc8, accd = _edge_call(tbl, ws_t, wd_t, idxc, zero8, zerod)
        x_gcn = acc8[0, :N].reshape(N, HID) + lp['gcnb']
        den = accd[1, :N, :HEADS]
        x_gat = (acc8[1, :N] / (den[..., None] + 1e-16)).reshape(N, HID) + lp['gatb']
        cur = _lrelu(_bn(x_gcn + x_gat, lp['bng'], lp['bnb']))
        base_gate = jax.nn.sigmoid(_lrelu(cur @ lp['gw1'] + lp['gb1']) @ lp['gw2'] + lp['gb2'])
        gf = jax.nn.sigmoid(_lrelu(gm @ lp['gmw1'] + lp['gmb1']) @ lp['gmw2'] + lp['gmb2'])
        depth = jax.nn.sigmoid(jnp.float32(i) / float(max(NL, 1)) * jnp.ones_like(base_gate))
        fg = base_gate * gf * depth
        cur = fg * cur + (1.0 - fg) * (prev + lp['denc'])
        prev = cur
        xs.append(cur)

    hcat = jnp.concatenate(xs, axis=1)
    h2 = _lrelu(hcat @ p['lin1w'] + p['lin1b'])
    m = _bn(_lrelu(h2 @ p['mw1'] + p['mb1']), p['mg1'], p['mbe1'])
    m = _bn(_lrelu(m @ p['mw2'] + p['mb2']), p['mg2'], p['mbe2'])
    raw = jax.nn.sigmoid(m @ p['mw3'] + p['mb3']).squeeze(-1)
    scaled = raw / p['temp'][0]
    sorted_desc = jnp.sort(scaled)[::-1]
    enhanced = jax.nn.softmax(scaled, axis=0)
    k = max(1, int(N * 0.2))
    thr = sorted_desc[k - 1]
    mask = (scaled >= thr).astype(jnp.float32)
    boosted = enhanced * (mask + 0.1 * (1.0 - mask))
    return boosted ** 3.0


# speedup vs baseline: 45.9613x; 1.3446x over previous
import functools

import jax
import jax.numpy as jnp
from jax import lax
from jax.experimental import pallas as pl
from jax.experimental.pallas import tpu as pltpu
from jax.experimental.pallas import tpu_sc as plsc

N = 10000
NP = 10240
RPT = NP // 16
HID = 128
HEADS = 8
DH = 16
NL = 4
HB = 9
B = 112
NBATCH = 96
EPT = B * NBATCH
EP = 16 * EPT
DB = 128

def _mesh():
    return plsc.VectorSubcoreMesh(
        core_axis_name="c", subcore_axis_name="s", num_cores=2, num_subcores=16)


_SC_PARAMS = pltpu.CompilerParams(use_tc_tiling_on_sc=False)


def _deg_body(ds_hbm, ones_hbm, zero_hbm, out_hbm, idx_v, ones_v, dacc):
    cid = lax.axis_index("c")
    sid = lax.axis_index("s")
    pltpu.sync_copy(zero_hbm.at[pl.ds(sid * RPT, RPT)], dacc.at[pl.ds(sid * RPT, RPT)])
    pltpu.sync_copy(ones_hbm, ones_v)
    plsc.subcore_barrier()
    base0 = (cid * 16 + sid) * (EP // 32)

    def batch(bi, carry):
        pltpu.sync_copy(ds_hbm.at[pl.ds(base0 + bi * DB, DB)], idx_v)
        pltpu.sync_copy(ones_v, dacc.at[idx_v], add=True)
        return carry

    lax.fori_loop(0, EP // 32 // DB, batch, 0)
    plsc.subcore_barrier()
    pltpu.sync_copy(dacc.at[pl.ds(sid * RPT, RPT)],
                    out_hbm.at[cid, pl.ds(sid * RPT, RPT)])


def _deg_call(*args):
    return pl.kernel(
        _deg_body,
        out_type=jax.ShapeDtypeStruct((2, NP, 16), jnp.float32),
        mesh=_mesh(),
        compiler_params=_SC_PARAMS,
        scratch_types=[
            pltpu.VMEM((DB,), jnp.int32),
            pltpu.VMEM((DB, 16), jnp.float32),
            pltpu.VMEM_SHARED((NP, 16), jnp.float32),
        ],
    )(*args)


def _edge_body(t_hbm, ws_hbm, wd_hbm, idx_hbm, zero8_hbm, zerod_hbm,
               out8_hbm, outd_hbm, ib0, ib1, r0, r1, s0, s1, d0, d1,
               isem0, isem1, gsem0, gsem1, acc8, accd):
    cid = lax.axis_index("c")
    sid = lax.axis_index("s")
    is_gcn = cid == 0
    ib = (ib0, ib1)
    rows = (r0, r1)
    wsv = (s0, s1)
    wdv = (d0, d1)
    isem = (isem0, isem1)
    gsem = (gsem0, gsem1)
    pltpu.sync_copy(zero8_hbm.at[pl.ds(sid * RPT, RPT)],
                    acc8.at[pl.ds(sid * RPT, RPT)])
    pltpu.sync_copy(zerod_hbm.at[pl.ds(sid * RPT, RPT)],
                    accd.at[pl.ds(sid * RPT, RPT)])

    def fire_i(i, b):
        pltpu.async_copy(idx_hbm.at[cid, sid, i], ib[b], isem[b])

    def wait_i(b):
        pltpu.make_async_copy(idx_hbm.at[cid, sid, 0], ib[b], isem[b]).wait()

    def fire_g(b):
        pltpu.async_copy(t_hbm.at[ib[b].at[0]], rows[b], gsem[b])
        pltpu.async_copy(ws_hbm.at[ib[b].at[0]], wsv[b], gsem[b])
        pltpu.async_copy(wd_hbm.at[ib[b].at[1]], wdv[b], gsem[b])

    def wait_g(b):
        pltpu.make_async_copy(t_hbm.at[ib[b].at[0]], rows[b], gsem[b]).wait()
        pltpu.make_async_copy(ws_hbm.at[ib[b].at[0]], wsv[b], gsem[b]).wait()
        pltpu.make_async_copy(wd_hbm.at[ib[b].at[1]], wdv[b], gsem[b]).wait()

    def compute(b):
        rv = rows[b]
        sv = wsv[b]
        dv = wdv[b]

        def edge_gcn(e, ecarry):
            w = sv[e] * dv[e]
            sv[e] = w
            wh = w[0]
            for h in range(HEADS):
                rv[e, pl.ds(16 * h, 16)] = rv[e, pl.ds(16 * h, 16)] * wh
            return ecarry

        def edge_gat(e, ecarry):
            s = sv[e] + dv[e]
            w = jnp.exp(jnp.where(s > 0.0, s, 0.2 * s))
            sv[e] = w
            for h in range(HEADS):
                rv[e, pl.ds(16 * h, 16)] = rv[e, pl.ds(16 * h, 16)] * w[h]
            return ecarry

        @pl.when(is_gcn)
        def _():
            lax.fori_loop(0, B, edge_gcn, 0, unroll=4)

        @pl.when(jnp.logical_not(is_gcn))
        def _():
            lax.fori_loop(0, B, edge_gat, 0, unroll=4)

    fire_i(0, 0)
    plsc.subcore_barrier()
    wait_i(0)
    fire_g(0)
    fire_i(1, 1)

    def batch_pair(ip, carry):
        for b in (0, 1):
            i = ip * 2 + b

            @pl.when(i + 1 <= NBATCH)
            def _():
                wait_i(1 - b)

            @pl.when(i + 1 < NBATCH)
            def _():
                fire_g(1 - b)

            wait_g(b)

            @pl.when(i + 2 <= NBATCH)
            def _():
                fire_i(i + 2, b)

            compute(b)
            pltpu.sync_copy(rows[b], acc8.at[ib[1 - b].at[2]], add=True)
            pltpu.sync_copy(wsv[b], accd.at[ib[1 - b].at[2]], add=True)
        return carry

    lax.fori_loop(0, NBATCH // 2, batch_pair, 0)
    plsc.subcore_barrier()
    pltpu.sync_copy(acc8.at[pl.ds(sid * RPT, RPT)],
                    out8_hbm.at[cid, pl.ds(sid * RPT, RPT)])
    pltpu.sync_copy(accd.at[pl.ds(sid * RPT, RPT)],
                    outd_hbm.at[cid, pl.ds(sid * RPT, RPT)])


def _edge_call(*args):
    return pl.kernel(
        _edge_body,
        out_type=(jax.ShapeDtypeStruct((2, NP, HID), jnp.float32),
                  jax.ShapeDtypeStruct((2, NP, 16), jnp.float32)),
        mesh=_mesh(),
        compiler_params=_SC_PARAMS,
        scratch_types=[
            pltpu.VMEM((3, B), jnp.int32),
            pltpu.VMEM((3, B), jnp.int32),
            pltpu.VMEM((B, HID), jnp.float32),
            pltpu.VMEM((B, HID), jnp.float32),
            pltpu.VMEM((B, 16), jnp.float32),
            pltpu.VMEM((B, 16), jnp.float32),
            pltpu.VMEM((B, 16), jnp.float32),
            pltpu.VMEM((B, 16), jnp.float32),
            pltpu.SemaphoreType.DMA,
            pltpu.SemaphoreType.DMA,
            pltpu.SemaphoreType.DMA,
            pltpu.SemaphoreType.DMA,
            pltpu.VMEM_SHARED((NP, HID), jnp.float32),
            pltpu.VMEM_SHARED((NP, 16), jnp.float32),
        ],
    )(*args)


def _lrelu(v):
    return jnp.where(v > 0, v, 0.2 * v)


_RB = 512
_NBLK = NP // _RB


def _fusion_body(xr, cw1, cb1, cg, cbe, ew1, eb1, eg, ebe, fw1, fb1, fg1, fbe1,
                 fw2, fb2, fg2, fbe2, faw1, fab1, faw2, fab2, daw, dab, bigw,
                 awm, h_out, t_out, a_out):
    xx = xr[...]
    cent = xx[:, :8]
    emb = xx[:, 8:]
    ce = _lrelu(jnp.dot(cent, cw1[...], preferred_element_type=jnp.float32)
                + cb1[...]) * cg[...] + cbe[...]
    ee = _lrelu(jnp.dot(emb, ew1[...], preferred_element_type=jnp.float32)
                + eb1[...]) * eg[...] + ebe[...]
    h = jnp.concatenate([ce, ee], axis=1)
    h = _lrelu(jnp.dot(h, fw1[...], preferred_element_type=jnp.float32)
               + fb1[...]) * fg1[...] + fbe1[...]
    h = _lrelu(jnp.dot(h, fw2[...], preferred_element_type=jnp.float32)
               + fb2[...]) * fg2[...] + fbe2[...]
    attw = jax.nn.sigmoid(
        jnp.dot(_lrelu(jnp.dot(h, faw1[...], preferred_element_type=jnp.float32)
                       + fab1[...]),
                faw2[...], preferred_element_type=jnp.float32) + fab2[...])
    h = h * attw
    h = jnp.dot(h, daw[...], preferred_element_type=jnp.float32) + dab[...]
    h_out[...] = h
    t_out[0] = jnp.dot(h, bigw[...][0], preferred_element_type=jnp.float32)
    t_out[1] = jnp.dot(h, bigw[...][1], preferred_element_type=jnp.float32)
    a_out[...] = jnp.dot(h, awm[...], preferred_element_type=jnp.float32)


def _wspec(shape):
    return pl.BlockSpec(shape, lambda i: tuple(0 for _ in shape))


def _fusion_call(xp, ws):
    rspec = pl.BlockSpec((_RB, 128), lambda i: (i, 0))
    return pl.pallas_call(
        _fusion_body,
        grid=(_NBLK,),
        in_specs=[rspec] + [_wspec(w.shape) for w in ws],
        out_specs=(rspec, pl.BlockSpec((2, _RB, 128), lambda i: (0, i, 0)),
                   pl.BlockSpec((_RB, 32), lambda i: (i, 0))),
        out_shape=(jax.ShapeDtypeStruct((NP, 128), jnp.float32),
                   jax.ShapeDtypeStruct((2, NP, 128), jnp.float32),
                   jax.ShapeDtypeStruct((NP, 32), jnp.float32)),
    )(xp, *ws)


def _layer_body(prev, accg, acca, den8, gcnb, gatb, bng, bnb, gw1, gb1, gw2,
                gb2, denc, rmat, gfd, bigw, awm, cur_out, t_out, a_out):
    xg = accg[...] + gcnb[...]
    denf = jnp.dot(den8[...], rmat[...], preferred_element_type=jnp.float32) + 1e-16
    xa = acca[...] / denf + gatb[...]
    cur0 = _lrelu((xg + xa) * bng[...] + bnb[...])
    g1 = _lrelu(jnp.dot(cur0, gw1[...], preferred_element_type=jnp.float32)
                + gb1[...])
    gate = jax.nn.sigmoid(jnp.dot(g1, gw2[...], preferred_element_type=jnp.float32)
                          + gb2[...])
    fg = gate * gfd[...][0, 0]
    cur = fg * cur0 + (1.0 - fg) * (prev[...] + denc[...])
    cur_out[...] = cur
    t_out[0] = jnp.dot(cur, bigw[...][0], preferred_element_type=jnp.float32)
    t_out[1] = jnp.dot(cur, bigw[...][1], preferred_element_type=jnp.float32)
    a_out[...] = jnp.dot(cur, awm[...], preferred_element_type=jnp.float32)


def _layer_call(prev, accg, acca, den8, ws):
    rspec = pl.BlockSpec((_RB, 128), lambda i: (i, 0))
    dspec = pl.BlockSpec((_RB, 16), lambda i: (i, 0))
    return pl.pallas_call(
        _layer_body,
        grid=(_NBLK,),
        in_specs=[rspec, rspec, rspec, dspec] + [_wspec(w.shape) for w in ws],
        out_specs=(rspec, pl.BlockSpec((2, _RB, 128), lambda i: (0, i, 0)),
                   pl.BlockSpec((_RB, 32), lambda i: (i, 0))),
        out_shape=(jax.ShapeDtypeStruct((NP, 128), jnp.float32),
                   jax.ShapeDtypeStruct((2, NP, 128), jnp.float32),
                   jax.ShapeDtypeStruct((NP, 32), jnp.float32)),
    )(prev, accg, acca, den8, *ws)


def _mlp_body(hcat, lin1w, lin1b, mw1, mb1, mg1, mbe1, mw2, mb2, mg2, mbe2,
              mw3, mb3, raw_out):
    h2 = _lrelu(jnp.dot(hcat[...], lin1w[...], preferred_element_type=jnp.float32)
                + lin1b[...])
    m = _lrelu(jnp.dot(h2, mw1[...], preferred_element_type=jnp.float32)
               + mb1[...]) * mg1[...] + mbe1[...]
    m = _lrelu(jnp.dot(m, mw2[...], preferred_element_type=jnp.float32)
               + mb2[...]) * mg2[...] + mbe2[...]
    raw_out[...] = jax.nn.sigmoid(
        jnp.dot(m, mw3[...], preferred_element_type=jnp.float32) + mb3[...])


def _mlp_call(hcat, ws):
    return pl.pallas_call(
        _mlp_body,
        grid=(_NBLK,),
        in_specs=[pl.BlockSpec((_RB, 512), lambda i: (i, 0))]
        + [_wspec(w.shape) for w in ws],
        out_specs=pl.BlockSpec((_RB, 1), lambda i: (i, 0)),
        out_shape=jax.ShapeDtypeStruct((NP, 1), jnp.float32),
    )(hcat, *ws)


_K20 = max(1, int(N * 0.2))


def _contrast_body(raw_ref, temp_ref, o_ref):
    rows, lanes = raw_ref.shape
    gidx = (lax.broadcasted_iota(jnp.int32, (rows, lanes), 0) * lanes
            + lax.broadcasted_iota(jnp.int32, (rows, lanes), 1))
    scaled = jnp.where(gidx < N, raw_ref[...] / temp_ref[...][0, 0], -1e30)
    bits = lax.bitcast_convert_type(scaled, jnp.int32)

    def step(_, carry):
        lo, hi = carry
        mid = lo + lax.shift_right_logical(hi - lo, 1)
        cnt = jnp.sum(jnp.where(bits >= mid, 1.0, 0.0))
        take = cnt >= float(_K20)
        return (jnp.where(take, mid, lo), jnp.where(take, hi, mid))

    lo, _ = lax.fori_loop(0, 31, step, (jnp.int32(0), jnp.int32(0x7F000000)))
    mask = (bits >= lo).astype(jnp.float32)
    m = jnp.max(scaled)
    ex = jnp.exp(scaled - m)
    den = jnp.sum(ex)
    boosted = (ex / den) * (mask + 0.1 * (1.0 - mask))
    o_ref[...] = boosted * boosted * boosted


def _contrast_call(raw2d, temp):
    return pl.pallas_call(
        _contrast_body,
        out_shape=jax.ShapeDtypeStruct(raw2d.shape, jnp.float32),
    )(raw2d, temp)


def kernel(x, edge_index, params):
    p = params
    E = edge_index.shape[1]
    num_nodes = min(float(N), 1000000.0)
    num_edges = min(float(E), 1000000.0)
    avg_deg = min(float(E) / max(float(N), 1.0), 1000.0)
    gm = jnp.array([[num_nodes, num_edges, avg_deg]], dtype=jnp.float32)
    bns = 1.0 / jnp.sqrt(jnp.float32(1.0 + 1e-5))

    def row(v):
        return v.reshape(1, -1)

    loop = jnp.arange(N, dtype=jnp.int32)
    esrc = jnp.concatenate([edge_index[0].astype(jnp.int32), loop])
    edst = jnp.concatenate([edge_index[1].astype(jnp.int32), loop])
    npad = EP - esrc.shape[0]
    src_p = jnp.pad(esrc, (0, npad))
    dst_p = jnp.pad(edst, (0, npad), constant_values=N)
    gs = jnp.stack([src_p, src_p + NP])
    gwd = jnp.stack([dst_p, dst_p + NP])
    gg = jnp.stack([gs.reshape(2, 16, NBATCH, B),
                    gwd.reshape(2, 16, NBATCH, B)], axis=3)
    gg = jnp.pad(gg, ((0, 0), (0, 0), (0, 1), (0, 0), (0, 0)))
    ds_r = dst_p.reshape(16, NBATCH, B)
    ds_sh = jnp.concatenate(
        [jnp.full((16, 1, B), N, jnp.int32), ds_r], axis=1)
    ds_sh = jnp.broadcast_to(ds_sh[None, :, :, None], (2, 16, NBATCH + 1, 1, B))
    idxc = jnp.concatenate([gg, ds_sh], axis=3)
    zero8 = jnp.zeros((NP, HID), jnp.float32)
    zerod = jnp.zeros((NP, 16), jnp.float32)
    ones16 = jnp.ones((DB, 16), jnp.float32)

    dout = _deg_call(dst_p, ones16, zerod)
    deg = dout[0, :N, 0] + dout[1, :N, 0]
    dinv = jnp.where(deg > 0, 1.0 / jnp.sqrt(deg), 0.0)
    dinv16 = jnp.broadcast_to(dinv[:, None], (N, 16))

    bigws = []
    awms = []
    for i in range(NL):
        lp = p['layers'][i]
        wsrc = (lp['gatw'].reshape(HID, HEADS, DH)
                * lp['gatas'][None]).sum(-1)
        wdst = (lp['gatw'].reshape(HID, HEADS, DH)
                * lp['gatad'][None]).sum(-1)
        z8 = jnp.zeros((HID, 8), jnp.float32)
        bigws.append(jnp.stack([lp['gcnw'], lp['gatw']]))
        awms.append(jnp.concatenate([wsrc, z8, wdst, z8], axis=1))

    rmat = jnp.zeros((16, HID), jnp.float32)
    rmat = rmat.at[jnp.repeat(jnp.arange(HEADS), DH),
                   jnp.arange(HID)].set(1.0)

    xp = jnp.pad(x, ((0, NP - N), (0, 0)))
    fws = [p['cw1'], row(p['cb1']), row(p['cg1'] * bns), row(p['cbe1']),
           p['ew1'], row(p['eb1']), row(p['eg1'] * bns), row(p['ebe1']),
           p['fw1'], row(p['fb1']), row(p['fg1'] * bns), row(p['fbe1']),
           p['fw2'], row(p['fb2']), row(p['fg2'] * bns), row(p['fbe2']),
           p['faw1'], row(p['fab1']), p['faw2'], row(p['fab2']),
           p['daw'], row(p['dab']), bigws[0], awms[0]]
    prev, t2, aw = _fusion_call(xp, fws)

    xs = []
    zpad = jnp.zeros((NP - N, 16), jnp.float32)
    for i in range(NL):
        lp = p['layers'][i]
        tblsc = t2.reshape(2 * NP, HID)
        ws_t = jnp.concatenate([dinv16, zpad, aw[:N, :16], zpad])
        wd_t = jnp.concatenate([dinv16, zpad, aw[:N, 16:], zpad])
        acc8, accd = _edge_call(tblsc, ws_t, wd_t, idxc, zero8, zerod)
        gf = jax.nn.sigmoid(
            _lrelu(gm @ lp['gmw1'] + lp['gmb1']) @ lp['gmw2'] + lp['gmb2'])
        depth = jax.nn.sigmoid(jnp.float32(i) / float(max(NL, 1)))
        gfd = (gf * depth).reshape(1, 1)
        lws = [row(lp['gcnb']), row(lp['gatb']), row(lp['bng'] * bns),
               row(lp['bnb']), lp['gw1'], row(lp['gb1']), lp['gw2'],
               row(lp['gb2']), lp['denc'], rmat, gfd,
               bigws[(i + 1) % NL], awms[(i + 1) % NL]]
        cur, t2, aw = _layer_call(prev, acc8[0], acc8[1], accd[1], lws)
        prev = cur
        xs.append(cur)

    hcat = jnp.concatenate(xs, axis=1)
    mws = [p['lin1w'], row(p['lin1b']),
           p['mw1'], row(p['mb1']), row(p['mg1'] * bns), row(p['mbe1']),
           p['mw2'], row(p['mb2']), row(p['mg2'] * bns), row(p['mbe2']),
           p['mw3'], row(p['mb3'])]
    raw = _mlp_call(hcat, mws)
    raw2d = raw.reshape(NP // 128, 128)
    boosted = _contrast_call(raw2d, p['temp'].reshape(1, 1))
    return boosted.reshape(-1)[:N]
